# Initial kernel scaffold; baseline (speedup 1.0000x reference)
import jax
import jax.numpy as jnp
from jax.experimental import pallas as pl


def kernel(x, edge_index, edge_attr, batch, W1, b1, W2, b2, Wl1, bl1, Wl2, bl2):
    raise NotImplementedError("write your pallas kernel here")


# speedup vs baseline: 7.6382x; 7.6382x over previous
import functools

import jax
import jax.numpy as jnp
from jax import lax
from jax.experimental import pallas as pl
from jax.experimental.pallas import tpu as pltpu
from jax.experimental.pallas import tpu_sc as plsc

N = 10000
E = 320000
G = 64
D_IN = 128
D_HID = 256
D_OUT = 64

CHUNK = 128
C1 = 79
E_PAD = 32 * C1 * CHUNK
C2 = E_PAD // (16 * CHUNK)
N_PAD = 10240
STRIPE = N_PAD // 16

_mesh = plsc.VectorSubcoreMesh(core_axis_name="c", subcore_axis_name="s")


def _zero_vmem(ref, nrows, width):
    def row(i, _):
        for k in range(width // 16):
            ref[i, pl.ds(k * 16, 16)] = jnp.zeros((16,), jnp.float32)
        return 0
    lax.fori_loop(0, nrows, row, 0)


DEG_W = 128


@functools.partial(
    pl.kernel,
    out_type=jax.ShapeDtypeStruct((2, N_PAD, DEG_W), jnp.float32),
    mesh=_mesh,
    scratch_types=[
        pltpu.VMEM((CHUNK,), jnp.int32),
        pltpu.VMEM((CHUNK,), jnp.float32),
        pltpu.VMEM((CHUNK, DEG_W), jnp.float32),
        pltpu.VMEM_SHARED((N_PAD, DEG_W), jnp.float32),
        pltpu.SemaphoreType.DMA,
    ],
)
def _sc_degree(dst_hbm, ew_hbm, out_hbm, dst_v, ew_v, msg_v, acc_sh, sem):
    c = lax.axis_index("c")
    s = lax.axis_index("s")
    w = s * 2 + c
    _zero_vmem(msg_v, CHUNK, DEG_W)
    for r in range(5):
        pltpu.sync_copy(msg_v, acc_sh.at[pl.ds(s * STRIPE + r * 128, 128)])
    plsc.subcore_barrier()

    def chunk(i, _):
        eb = pl.multiple_of(w * (C1 * CHUNK) + i * CHUNK, CHUNK)
        pltpu.sync_copy(dst_hbm.at[pl.ds(eb, CHUNK)], dst_v)
        pltpu.sync_copy(ew_hbm.at[pl.ds(eb, CHUNK)], ew_v)
        def grp(j, _):
            ew16 = ew_v[pl.ds(16 * j, 16)]
            for i16 in range(16):
                for k in range(DEG_W // 16):
                    msg_v[16 * j + i16, pl.ds(16 * k, 16)] = (
                        jnp.broadcast_to(ew16[i16], (16,)))
            return 0
        lax.fori_loop(0, CHUNK // 16, grp, 0)
        pltpu.sync_copy(msg_v, acc_sh.at[dst_v], add=True)
        return 0
    lax.fori_loop(0, C1, chunk, 0)
    plsc.subcore_barrier()
    pltpu.sync_copy(acc_sh.at[pl.ds(s * STRIPE, STRIPE)],
                    out_hbm.at[c, pl.ds(s * STRIPE, STRIPE)])


@functools.partial(
    pl.kernel,
    out_type=jax.ShapeDtypeStruct((2, N_PAD, D_IN), jnp.float32),
    mesh=_mesh,
    scratch_types=[
        pltpu.VMEM((CHUNK,), jnp.int32),
        pltpu.VMEM((CHUNK,), jnp.int32),
        pltpu.VMEM((CHUNK,), jnp.float32),
        pltpu.VMEM((CHUNK, D_IN), jnp.float32),
        pltpu.VMEM_SHARED((N_PAD, D_IN), jnp.float32),
        pltpu.SemaphoreType.DMA,
    ],
)
def _sc_prop1(src_hbm, dst_hbm, ew_hbm, xs_hbm, out_hbm,
              src_v, dst_v, ew_v, rows_v, acc_sh, sem):
    c = lax.axis_index("c")
    s = lax.axis_index("s")
    w = s * 2 + c
    _zero_vmem(rows_v, 128, D_IN)
    for r in range(5):
        pltpu.sync_copy(rows_v, acc_sh.at[pl.ds(s * STRIPE + r * 128, 128)])
    plsc.subcore_barrier()

    def chunk(i, _):
        eb = pl.multiple_of(w * (C1 * CHUNK) + i * CHUNK, CHUNK)
        pltpu.sync_copy(src_hbm.at[pl.ds(eb, CHUNK)], src_v)
        pltpu.sync_copy(dst_hbm.at[pl.ds(eb, CHUNK)], dst_v)
        pltpu.sync_copy(ew_hbm.at[pl.ds(eb, CHUNK)], ew_v)
        pltpu.async_copy(xs_hbm.at[src_v], rows_v, sem).wait()
        def scale(j, _):
            ew16 = ew_v[pl.ds(16 * j, 16)]
            for i in range(16):
                wt = ew16[i]
                e = 16 * j + i
                for k in range(D_IN // 16):
                    sl = pl.ds(16 * k, 16)
                    rows_v[e, sl] = rows_v[e, sl] * wt
            return 0
        lax.fori_loop(0, CHUNK // 16, scale, 0)
        pltpu.sync_copy(rows_v, acc_sh.at[dst_v], add=True)
        return 0
    lax.fori_loop(0, C1, chunk, 0)
    plsc.subcore_barrier()
    pltpu.sync_copy(acc_sh.at[pl.ds(s * STRIPE, STRIPE)],
                    out_hbm.at[c, pl.ds(s * STRIPE, STRIPE)])


@functools.partial(
    pl.kernel,
    out_type=jax.ShapeDtypeStruct((2, N_PAD, D_IN), jnp.float32),
    mesh=_mesh,
    scratch_types=[
        pltpu.VMEM((CHUNK,), jnp.int32),
        pltpu.VMEM((CHUNK,), jnp.int32),
        pltpu.VMEM((CHUNK,), jnp.float32),
        pltpu.VMEM((CHUNK, D_IN), jnp.float32),
        pltpu.VMEM_SHARED((N_PAD, D_IN), jnp.float32),
        pltpu.SemaphoreType.DMA,
    ],
)
def _sc_prop2(src_hbm, dst_hbm, ew_hbm, hs_hbm, out_hbm,
              src_v, dst_v, ew_v, rows_v, acc_sh, sem):
    c = lax.axis_index("c")
    s = lax.axis_index("s")
    _zero_vmem(rows_v, 128, D_IN)
    for r in range(5):
        pltpu.sync_copy(rows_v, acc_sh.at[pl.ds(s * STRIPE + r * 128, 128)])
    plsc.subcore_barrier()

    def chunk(i, _):
        eb = pl.multiple_of(s * (C2 * CHUNK) + i * CHUNK, CHUNK)
        pltpu.sync_copy(src_hbm.at[pl.ds(eb, CHUNK)], src_v)
        pltpu.sync_copy(dst_hbm.at[pl.ds(eb, CHUNK)], dst_v)
        pltpu.sync_copy(ew_hbm.at[pl.ds(eb, CHUNK)], ew_v)
        off = c * N
        for k in range(CHUNK // 16):
            sl = pl.ds(16 * k, 16)
            src_v[sl] = src_v[sl] + off
        pltpu.async_copy(hs_hbm.at[src_v], rows_v, sem).wait()
        def scale(j, _):
            ew16 = ew_v[pl.ds(16 * j, 16)]
            for i in range(16):
                wt = ew16[i]
                e = 16 * j + i
                for k in range(D_IN // 16):
                    sl = pl.ds(16 * k, 16)
                    rows_v[e, sl] = rows_v[e, sl] * wt
            return 0
        lax.fori_loop(0, CHUNK // 16, scale, 0)
        pltpu.sync_copy(rows_v, acc_sh.at[dst_v], add=True)
        return 0
    lax.fori_loop(0, C2, chunk, 0)
    plsc.subcore_barrier()
    pltpu.sync_copy(acc_sh.at[pl.ds(s * STRIPE, STRIPE)],
                    out_hbm.at[c, pl.ds(s * STRIPE, STRIPE)])


def _tc_prep_body(deg_ref, x_ref, dis_ref, xs_ref):
    d = deg_ref[0, 0:N, 0:1] + deg_ref[1, 0:N, 0:1] + 1.0
    dis = jnp.where(d > 0, lax.rsqrt(d), 0.0)
    dis_ref[...] = dis
    xs_ref[...] = x_ref[...] * dis


def _tc_prep(deg, x):
    return pl.pallas_call(
        _tc_prep_body,
        out_shape=[jax.ShapeDtypeStruct((N, 1), jnp.float32),
                   jax.ShapeDtypeStruct((N, D_IN), jnp.float32)],
    )(deg, x)


R_BLK = 1000


def _tc_layer1_body(acc_ref, xs_ref, dis_ref, w1_ref, b1_ref, hs_ref):
    dis = dis_ref[...]
    p = (acc_ref[0] + acc_ref[1] + xs_ref[...]) * dis
    h = jnp.maximum(jnp.dot(p, w1_ref[...],
                            preferred_element_type=jnp.float32) + b1_ref[...], 0.0)
    hs = h * dis
    hs_ref[0] = hs[:, :D_IN]
    hs_ref[1] = hs[:, D_IN:]


def _tc_layer1(acc, xs, dis, W1, b1):
    grid = N // R_BLK
    return pl.pallas_call(
        _tc_layer1_body,
        grid=(grid,),
        in_specs=[
            pl.BlockSpec((2, R_BLK, D_IN), lambda i: (0, i, 0)),
            pl.BlockSpec((R_BLK, D_IN), lambda i: (i, 0)),
            pl.BlockSpec((R_BLK, 1), lambda i: (i, 0)),
            pl.BlockSpec((D_IN, D_HID), lambda i: (0, 0)),
            pl.BlockSpec((1, D_HID), lambda i: (0, 0)),
        ],
        out_specs=pl.BlockSpec((2, R_BLK, D_IN), lambda i: (0, i, 0)),
        out_shape=jax.ShapeDtypeStruct((2, N, D_IN), jnp.float32),
    )(acc, xs, dis, W1, b1)


def _tc_layer2_body(acc_ref, hs_ref, dis_ref, w2_ref, b2_ref,
                    batch_ref, sums_ref, cnts_ref):
    i = pl.program_id(0)
    dis = dis_ref[...]
    p0 = (acc_ref[0] + hs_ref[0]) * dis
    p1 = (acc_ref[1] + hs_ref[1]) * dis
    h2 = jnp.dot(p0, w2_ref[:D_IN, :], preferred_element_type=jnp.float32)
    h2 = h2 + jnp.dot(p1, w2_ref[D_IN:, :], preferred_element_type=jnp.float32)
    h2 = jnp.maximum(h2 + b2_ref[...], 0.0)
    ids = batch_ref[...].reshape(1, R_BLK)
    gids = lax.broadcasted_iota(jnp.int32, (G, R_BLK), 0)
    onehot = jnp.where(ids == gids, 1.0, 0.0)

    @pl.when(i == 0)
    def _():
        sums_ref[...] = jnp.zeros_like(sums_ref)
        cnts_ref[...] = jnp.zeros_like(cnts_ref)

    sums_ref[...] += jnp.dot(onehot, h2, preferred_element_type=jnp.float32)
    cnt = jnp.sum(onehot, axis=1, keepdims=True)
    cnts_ref[...] += jnp.broadcast_to(cnt, (G, 128))


def _tc_layer2(acc2, hs, dis, W2, b2, batch2d):
    grid = N // R_BLK
    return pl.pallas_call(
        _tc_layer2_body,
        grid=(grid,),
        in_specs=[
            pl.BlockSpec((2, R_BLK, D_IN), lambda i: (0, i, 0)),
            pl.BlockSpec((2, R_BLK, D_IN), lambda i: (0, i, 0)),
            pl.BlockSpec((R_BLK, 1), lambda i: (i, 0)),
            pl.BlockSpec((D_HID, D_HID), lambda i: (0, 0)),
            pl.BlockSpec((1, D_HID), lambda i: (0, 0)),
            pl.BlockSpec((R_BLK, 1), lambda i: (i, 0)),
        ],
        out_specs=[
            pl.BlockSpec((G, D_HID), lambda i: (0, 0)),
            pl.BlockSpec((G, 128), lambda i: (0, 0)),
        ],
        out_shape=[jax.ShapeDtypeStruct((G, D_HID), jnp.float32),
                   jax.ShapeDtypeStruct((G, 128), jnp.float32)],
    )(acc2, hs, dis, W2, b2, batch2d)


def _tc_head_body(sums_ref, cnts_ref, wl1_ref, bl1_ref, wl2_ref, bl2_ref, out_ref):
    g = sums_ref[...] / jnp.maximum(cnts_ref[:, 0:1], 1.0)
    a = jnp.maximum(jnp.dot(g, wl1_ref[...],
                            preferred_element_type=jnp.float32) + bl1_ref[...], 0.0)
    out_ref[...] = jnp.dot(a, wl2_ref[...],
                           preferred_element_type=jnp.float32) + bl2_ref[...]


def _tc_head(sums, cnts, Wl1, bl1, Wl2, bl2):
    return pl.pallas_call(
        _tc_head_body,
        out_shape=jax.ShapeDtypeStruct((G, D_OUT), jnp.float32),
    )(sums, cnts, Wl1, bl1, Wl2, bl2)


def kernel(x, edge_index, edge_attr, batch, W1, b1, W2, b2, Wl1, bl1, Wl2, bl2):
    pad = E_PAD - E
    src = jnp.concatenate([edge_index[0], jnp.zeros((pad,), jnp.int32)])
    dst = jnp.concatenate([edge_index[1], jnp.zeros((pad,), jnp.int32)])
    ew = jnp.concatenate([edge_attr, jnp.zeros((pad,), jnp.float32)])
    batch2d = batch.reshape(N, 1)
    b1r = b1.reshape(1, D_HID)
    b2r = b2.reshape(1, D_HID)
    bl1r = bl1.reshape(1, D_IN)
    bl2r = bl2.reshape(1, D_OUT)

    deg = _sc_degree(dst, ew)
    dis, xs = _tc_prep(deg, x)
    acc1 = _sc_prop1(src, dst, ew, xs)
    hs = _tc_layer1(acc1, xs, dis, W1, b1r)
    acc2 = _sc_prop2(src, dst, ew, hs.reshape(2 * N, D_IN))
    sums, cnts = _tc_layer2(acc2, hs, dis, W2, b2r, batch2d)
    return _tc_head(sums, cnts, Wl1, bl1, Wl2, bl2)


# speedup vs baseline: 9.1483x; 1.1977x over previous
import functools

import jax
import jax.numpy as jnp
from jax import lax
from jax.experimental import pallas as pl
from jax.experimental.pallas import tpu as pltpu
from jax.experimental.pallas import tpu_sc as plsc

N = 10000
E = 320000
G = 64
D_IN = 128
D_HID = 256
D_OUT = 64

CHUNK = 128
C1 = 80
E_PAD = 32 * C1 * CHUNK
C2 = E_PAD // (16 * CHUNK)
N_PAD = 10240
STRIPE = N_PAD // 16

_mesh = plsc.VectorSubcoreMesh(core_axis_name="c", subcore_axis_name="s")


def _zero_vmem(ref, nrows, width):
    def row(i, _):
        for k in range(width // 16):
            ref[i, pl.ds(k * 16, 16)] = jnp.zeros((16,), jnp.float32)
        return 0
    lax.fori_loop(0, nrows, row, 0)


DEG_W = 128
SB = 40


@functools.partial(
    pl.kernel,
    out_type=jax.ShapeDtypeStruct((2, N_PAD, DEG_W), jnp.float32),
    mesh=_mesh,
    scratch_types=[
        pltpu.VMEM((SB, CHUNK), jnp.int32),
        pltpu.VMEM((SB, CHUNK), jnp.float32),
        pltpu.VMEM((CHUNK, DEG_W), jnp.float32),
        pltpu.VMEM((CHUNK, DEG_W), jnp.float32),
        pltpu.VMEM_SHARED((N_PAD, DEG_W), jnp.float32),
        pltpu.SemaphoreType.DMA,
        pltpu.SemaphoreType.DMA,
    ],
)
def _sc_degree(dst2_hbm, ew2_hbm, out_hbm, dst2_v, ew2_v, msg0, msg1,
               acc_sh, ssem0, ssem1):
    c = lax.axis_index("c")
    s = lax.axis_index("s")
    cb = pl.multiple_of((s * 2 + c) * C1, 8)
    _zero_vmem(msg0, CHUNK, DEG_W)
    for r in range(5):
        pltpu.sync_copy(msg0, acc_sh.at[pl.ds(s * STRIPE + r * 128, 128)])
    plsc.subcore_barrier()

    def buildmsg(msg, r):
        def grp(j, _):
            ew16 = ew2_v[r, pl.ds(16 * j, 16)]
            for i16 in range(16):
                for k in range(DEG_W // 16):
                    msg[16 * j + i16, pl.ds(16 * k, 16)] = (
                        jnp.broadcast_to(ew16[i16], (16,)))
            return 0
        lax.fori_loop(0, CHUNK // 16, grp, 0)

    def stage(t, _):
        so = pl.multiple_of(cb + t * SB, 8)
        pltpu.sync_copy(dst2_hbm.at[pl.ds(so, SB)], dst2_v)
        pltpu.sync_copy(ew2_hbm.at[pl.ds(so, SB)], ew2_v)

        def pair(i, _):
            a = 2 * i
            b = a + 1
            @pl.when(i > 0)
            def _():
                pltpu.make_async_copy(msg0, acc_sh.at[dst2_v.at[a]], ssem0).wait()
            buildmsg(msg0, a)
            pltpu.async_copy(msg0, acc_sh.at[dst2_v.at[a]], ssem0, add=True)
            @pl.when(i > 0)
            def _():
                pltpu.make_async_copy(msg1, acc_sh.at[dst2_v.at[b]], ssem1).wait()
            buildmsg(msg1, b)
            pltpu.async_copy(msg1, acc_sh.at[dst2_v.at[b]], ssem1, add=True)
            return 0
        lax.fori_loop(0, SB // 2, pair, 0)
        pltpu.make_async_copy(msg0, acc_sh.at[dst2_v.at[0]], ssem0).wait()
        pltpu.make_async_copy(msg1, acc_sh.at[dst2_v.at[0]], ssem1).wait()
        return 0
    lax.fori_loop(0, C1 // SB, stage, 0)
    plsc.subcore_barrier()
    pltpu.sync_copy(acc_sh.at[pl.ds(s * STRIPE, STRIPE)],
                    out_hbm.at[c, pl.ds(s * STRIPE, STRIPE)])


def _make_prop(chunks, split32):
    @functools.partial(
        pl.kernel,
        out_type=jax.ShapeDtypeStruct((2, N_PAD, D_IN), jnp.float32),
        mesh=_mesh,
        scratch_types=[
            pltpu.VMEM((SB, CHUNK), jnp.int32),
            pltpu.VMEM((SB, CHUNK), jnp.int32),
            pltpu.VMEM((SB, CHUNK), jnp.float32),
            pltpu.VMEM((CHUNK,), jnp.int32),
            pltpu.VMEM((CHUNK,), jnp.int32),
            pltpu.VMEM((CHUNK, D_IN), jnp.float32),
            pltpu.VMEM((CHUNK, D_IN), jnp.float32),
            pltpu.VMEM_SHARED((N_PAD, D_IN), jnp.float32),
            pltpu.SemaphoreType.DMA,
            pltpu.SemaphoreType.DMA,
        ],
    )
    def prop(src2_hbm, dst2_hbm, ew2_hbm, table_hbm, out_hbm,
             src2_v, dst2_v, ew2_v, gidx0, gidx1, buf0, buf1,
             acc_sh, gsem0, gsem1):
        c = lax.axis_index("c")
        s = lax.axis_index("s")
        if split32:
            cb = (s * 2 + c) * chunks
            off = 0
        else:
            cb = s * chunks
            off = c * N
        cb = pl.multiple_of(cb, 8)
        _zero_vmem(buf0, CHUNK, D_IN)
        for r in range(5):
            pltpu.sync_copy(buf0, acc_sh.at[pl.ds(s * STRIPE + r * 128, 128)])
        plsc.subcore_barrier()

        def build(gidx, r):
            for k in range(CHUNK // 16):
                sl = pl.ds(16 * k, 16)
                gidx[sl] = src2_v[r, sl] + off

        def scale(buf, r):
            def grp(j, _):
                ew16 = ew2_v[r, pl.ds(16 * j, 16)]
                for i16 in range(16):
                    wt = ew16[i16]
                    e = 16 * j + i16
                    for k in range(D_IN // 16):
                        sl = pl.ds(16 * k, 16)
                        buf[e, sl] = buf[e, sl] * wt
                return 0
            lax.fori_loop(0, CHUNK // 16, grp, 0)

        def stage(t, _):
            so = pl.multiple_of(cb + t * SB, 8)
            pltpu.sync_copy(src2_hbm.at[pl.ds(so, SB)], src2_v)
            pltpu.sync_copy(dst2_hbm.at[pl.ds(so, SB)], dst2_v)
            pltpu.sync_copy(ew2_hbm.at[pl.ds(so, SB)], ew2_v)
            build(gidx0, 0)
            pltpu.async_copy(table_hbm.at[gidx0], buf0, gsem0)

            def pair(i, _):
                a = 2 * i
                b = a + 1
                build(gidx1, b)
                pltpu.async_copy(table_hbm.at[gidx1], buf1, gsem1)
                pltpu.make_async_copy(table_hbm.at[gidx0], buf0, gsem0).wait()
                scale(buf0, a)
                pltpu.sync_copy(buf0, acc_sh.at[dst2_v.at[a]], add=True)
                @pl.when(b + 1 < SB)
                def _():
                    build(gidx0, b + 1)
                    pltpu.async_copy(table_hbm.at[gidx0], buf0, gsem0)
                pltpu.make_async_copy(table_hbm.at[gidx1], buf1, gsem1).wait()
                scale(buf1, b)
                pltpu.sync_copy(buf1, acc_sh.at[dst2_v.at[b]], add=True)
                return 0
            lax.fori_loop(0, SB // 2, pair, 0)
            return 0
        lax.fori_loop(0, chunks // SB, stage, 0)
        plsc.subcore_barrier()
        pltpu.sync_copy(acc_sh.at[pl.ds(s * STRIPE, STRIPE)],
                        out_hbm.at[c, pl.ds(s * STRIPE, STRIPE)])
    return prop


_sc_prop1 = _make_prop(C1, True)
_sc_prop2 = _make_prop(C2, False)


def _tc_prep_body(deg_ref, x_ref, dis_ref, xs_ref):
    d = deg_ref[0, 0:N, 0:1] + deg_ref[1, 0:N, 0:1] + 1.0
    dis = jnp.where(d > 0, lax.rsqrt(d), 0.0)
    dis_ref[...] = dis
    xs_ref[...] = x_ref[...] * dis


def _tc_prep(deg, x):
    return pl.pallas_call(
        _tc_prep_body,
        out_shape=[jax.ShapeDtypeStruct((N, 1), jnp.float32),
                   jax.ShapeDtypeStruct((N, D_IN), jnp.float32)],
    )(deg, x)


R_BLK = 1000


def _tc_layer1_body(acc_ref, xs_ref, dis_ref, w1_ref, b1_ref, hs_ref):
    dis = dis_ref[...]
    p = (acc_ref[0] + acc_ref[1] + xs_ref[...]) * dis
    h = jnp.maximum(jnp.dot(p, w1_ref[...],
                            preferred_element_type=jnp.float32) + b1_ref[...], 0.0)
    hs = h * dis
    hs_ref[0] = hs[:, :D_IN]
    hs_ref[1] = hs[:, D_IN:]


def _tc_layer1(acc, xs, dis, W1, b1):
    grid = N // R_BLK
    return pl.pallas_call(
        _tc_layer1_body,
        grid=(grid,),
        in_specs=[
            pl.BlockSpec((2, R_BLK, D_IN), lambda i: (0, i, 0)),
            pl.BlockSpec((R_BLK, D_IN), lambda i: (i, 0)),
            pl.BlockSpec((R_BLK, 1), lambda i: (i, 0)),
            pl.BlockSpec((D_IN, D_HID), lambda i: (0, 0)),
            pl.BlockSpec((1, D_HID), lambda i: (0, 0)),
        ],
        out_specs=pl.BlockSpec((2, R_BLK, D_IN), lambda i: (0, i, 0)),
        out_shape=jax.ShapeDtypeStruct((2, N, D_IN), jnp.float32),
    )(acc, xs, dis, W1, b1)


def _tc_layer2_body(acc_ref, hs_ref, dis_ref, w2_ref, b2_ref,
                    batch_ref, sums_ref, cnts_ref):
    i = pl.program_id(0)
    dis = dis_ref[...]
    p0 = (acc_ref[0] + hs_ref[0]) * dis
    p1 = (acc_ref[1] + hs_ref[1]) * dis
    h2 = jnp.dot(p0, w2_ref[:D_IN, :], preferred_element_type=jnp.float32)
    h2 = h2 + jnp.dot(p1, w2_ref[D_IN:, :], preferred_element_type=jnp.float32)
    h2 = jnp.maximum(h2 + b2_ref[...], 0.0)
    ids = batch_ref[...].reshape(1, R_BLK)
    gids = lax.broadcasted_iota(jnp.int32, (G, R_BLK), 0)
    onehot = jnp.where(ids == gids, 1.0, 0.0)

    @pl.when(i == 0)
    def _():
        sums_ref[...] = jnp.zeros_like(sums_ref)
        cnts_ref[...] = jnp.zeros_like(cnts_ref)

    sums_ref[...] += jnp.dot(onehot, h2, preferred_element_type=jnp.float32)
    cnt = jnp.sum(onehot, axis=1, keepdims=True)
    cnts_ref[...] += jnp.broadcast_to(cnt, (G, 128))


def _tc_layer2(acc2, hs, dis, W2, b2, batch2d):
    grid = N // R_BLK
    return pl.pallas_call(
        _tc_layer2_body,
        grid=(grid,),
        in_specs=[
            pl.BlockSpec((2, R_BLK, D_IN), lambda i: (0, i, 0)),
            pl.BlockSpec((2, R_BLK, D_IN), lambda i: (0, i, 0)),
            pl.BlockSpec((R_BLK, 1), lambda i: (i, 0)),
            pl.BlockSpec((D_HID, D_HID), lambda i: (0, 0)),
            pl.BlockSpec((1, D_HID), lambda i: (0, 0)),
            pl.BlockSpec((R_BLK, 1), lambda i: (i, 0)),
        ],
        out_specs=[
            pl.BlockSpec((G, D_HID), lambda i: (0, 0)),
            pl.BlockSpec((G, 128), lambda i: (0, 0)),
        ],
        out_shape=[jax.ShapeDtypeStruct((G, D_HID), jnp.float32),
                   jax.ShapeDtypeStruct((G, 128), jnp.float32)],
    )(acc2, hs, dis, W2, b2, batch2d)


def _tc_head_body(sums_ref, cnts_ref, wl1_ref, bl1_ref, wl2_ref, bl2_ref, out_ref):
    g = sums_ref[...] / jnp.maximum(cnts_ref[:, 0:1], 1.0)
    a = jnp.maximum(jnp.dot(g, wl1_ref[...],
                            preferred_element_type=jnp.float32) + bl1_ref[...], 0.0)
    out_ref[...] = jnp.dot(a, wl2_ref[...],
                           preferred_element_type=jnp.float32) + bl2_ref[...]


def _tc_head(sums, cnts, Wl1, bl1, Wl2, bl2):
    return pl.pallas_call(
        _tc_head_body,
        out_shape=jax.ShapeDtypeStruct((G, D_OUT), jnp.float32),
    )(sums, cnts, Wl1, bl1, Wl2, bl2)


def kernel(x, edge_index, edge_attr, batch, W1, b1, W2, b2, Wl1, bl1, Wl2, bl2):
    pad = E_PAD - E
    src = jnp.concatenate([edge_index[0], jnp.zeros((pad,), jnp.int32)])
    dst = jnp.concatenate([edge_index[1], jnp.zeros((pad,), jnp.int32)])
    ew = jnp.concatenate([edge_attr, jnp.zeros((pad,), jnp.float32)])
    src = src.reshape(E_PAD // CHUNK, CHUNK)
    dst = dst.reshape(E_PAD // CHUNK, CHUNK)
    ew = ew.reshape(E_PAD // CHUNK, CHUNK)
    batch2d = batch.reshape(N, 1)
    b1r = b1.reshape(1, D_HID)
    b2r = b2.reshape(1, D_HID)
    bl1r = bl1.reshape(1, D_IN)
    bl2r = bl2.reshape(1, D_OUT)

    deg = _sc_degree(dst, ew)
    dis, xs = _tc_prep(deg, x)
    acc1 = _sc_prop1(src, dst, ew, xs)
    hs = _tc_layer1(acc1, xs, dis, W1, b1r)
    acc2 = _sc_prop2(src, dst, ew, hs.reshape(2 * N, D_IN))
    sums, cnts = _tc_layer2(acc2, hs, dis, W2, b2r, batch2d)
    return _tc_head(sums, cnts, Wl1, bl1, Wl2, bl2)
